# baseline (speedup 1.0000x reference)
import functools

import jax
import jax.numpy as jnp
from jax import lax
from jax.experimental import pallas as pl
from jax.experimental.pallas import tpu as pltpu
from jax.experimental.pallas import tpu_sc as plsc

_LW_BITS = 17
_LW = 1 << _LW_BITS
_CK_BITS = 12
_CK = 1 << _CK_BITS


def _detile_body(x_ref, o_ref):
    o_ref[...] = x_ref[...].reshape(8 * _LW)


def _detile_tc(xt, strips, n_windows):
    n_strips = len(strips)
    base = strips[0]
    return pl.pallas_call(
        _detile_body,
        grid=(n_strips, n_windows),
        in_specs=[pl.BlockSpec((8, _LW),
                               lambda g, w: (g + jnp.int32(base), w))],
        out_specs=pl.BlockSpec((8 * _LW,), lambda g, w: (g * n_windows + w,)),
        out_shape=jax.ShapeDtypeStruct(
            (n_strips * n_windows * 8 * _LW,), jnp.float32),
    )(xt)


def _detile_sc(xt, strips, n_rows):
    n_strips = len(strips)
    base = strips[0]
    n_windows = 8
    full_chunks = _LW // _CK
    tail = n_rows - (n_windows - 1) * _LW
    tail_chunks = tail // _CK
    tail_rem = 512

    info = plsc.get_sparse_core_info()
    mesh = plsc.VectorSubcoreMesh(core_axis_name="c", subcore_axis_name="s")

    @functools.partial(
        pl.kernel,
        mesh=mesh,
        out_type=jax.ShapeDtypeStruct(
            (n_strips * n_windows * 8 * _LW,), jnp.float32),
        scratch_types=[
            pltpu.VMEM((8, _CK), jnp.float32),
            pltpu.VMEM((8, _CK), jnp.float32),
            pltpu.VMEM((8, tail_rem), jnp.float32),
            pltpu.SemaphoreType.DMA,
            pltpu.SemaphoreType.DMA,
        ],
    )
    def detile_kernel(xt_hbm, out_hbm, vb0, vb1, vbt, sem_in, sem_out):
        wid = lax.axis_index("s") * info.num_cores + lax.axis_index("c")
        g_local = lax.shift_right_logical(wid, jnp.int32(3))
        w = wid & jnp.int32(7)
        strip_row = (jnp.int32(base) + g_local) * jnp.int32(8)
        lane0 = w * jnp.int32(_LW)
        dst_win = (g_local * jnp.int32(n_windows) + w) * jnp.int32(8 * _LW)
        nchunks = jnp.where(w == jnp.int32(n_windows - 1),
                            jnp.int32(tail_chunks), jnp.int32(full_chunks))
        vbs = (vb0, vb1)

        def read_chunk(m, vb):
            pltpu.async_copy(
                xt_hbm.at[pl.ds(strip_row, 8),
                          pl.ds(lane0 + m * jnp.int32(_CK), _CK)],
                vb, sem_in)

        def read_wait(m, vb):
            pltpu.make_async_copy(
                xt_hbm.at[pl.ds(strip_row, 8),
                          pl.ds(lane0 + m * jnp.int32(_CK), _CK)],
                vb, sem_in).wait()

        def write_chunk(m, vb):
            for r in range(8):
                pltpu.async_copy(
                    vb.at[jnp.int32(r)],
                    out_hbm.at[pl.ds(
                        dst_win + (jnp.int32(r * 32) + m) * jnp.int32(_CK),
                        _CK)],
                    sem_out)

        def write_wait(m, vb):
            for r in range(8):
                pltpu.make_async_copy(
                    vb.at[jnp.int32(r)],
                    out_hbm.at[pl.ds(
                        dst_win + (jnp.int32(r * 32) + m) * jnp.int32(_CK),
                        _CK)],
                    sem_out).wait()

        read_chunk(jnp.int32(0), vbs[0])

        def body(_, m):
            even = (m & jnp.int32(1)) == jnp.int32(0)

            def run(vb, vb_next):
                read_wait(m, vb)

                @pl.when(m + jnp.int32(1) < nchunks)
                def _():
                    read_chunk(m + jnp.int32(1), vb_next)

                write_chunk(m, vb)
                write_wait(m, vb)

            @pl.when(even)
            def _():
                run(vbs[0], vbs[1])

            @pl.when(jnp.logical_not(even))
            def _():
                run(vbs[1], vbs[0])

            return m + jnp.int32(1)

        lax.fori_loop(0, nchunks, body, jnp.int32(0))

        @pl.when(w == jnp.int32(n_windows - 1))
        def _():
            rem0 = lane0 + jnp.int32(tail_chunks * _CK)
            pltpu.sync_copy(
                xt_hbm.at[pl.ds(strip_row, 8), pl.ds(rem0, tail_rem)], vbt)
            for r in range(8):
                pltpu.sync_copy(
                    vbt.at[jnp.int32(r)],
                    out_hbm.at[pl.ds(
                        dst_win + jnp.int32(r * 32 + tail_chunks) *
                        jnp.int32(_CK),
                        tail_rem)])

    return detile_kernel(xt)


def _sc_gather(scratch_lo, scratch_hi, idx1d, n_cols, b):
    info = plsc.get_sparse_core_info()
    num_workers = info.num_cores * info.num_subcores
    lanes = info.num_lanes
    cols_per_tile = n_cols // num_workers
    half = num_workers // 2

    mesh = plsc.VectorSubcoreMesh(core_axis_name="c", subcore_axis_name="s")

    @functools.partial(
        pl.kernel,
        mesh=mesh,
        out_type=jax.ShapeDtypeStruct((n_cols * b,), jnp.float32),
        scratch_types=(
            [pltpu.VMEM((b,), jnp.int32) for _ in range(cols_per_tile)]
            + [pltpu.VMEM((b,), jnp.float32) for _ in range(cols_per_tile)]
            + [pltpu.SemaphoreType.DMA]
        ),
    )
    def gather_kernel(lo_hbm, hi_hbm, idx_hbm, out_hbm, *refs):
        idx_vs = refs[:cols_per_tile]
        val_vs = refs[cols_per_tile:2 * cols_per_tile]
        sem = refs[2 * cols_per_tile]
        wid = lax.axis_index("s") * info.num_cores + lax.axis_index("c")
        is_lo = wid < jnp.int32(half)
        mask_lw = jnp.int32(_LW - 1)
        mask_ck = jnp.int32(_CK - 1)

        local = jnp.where(is_lo, wid, wid - jnp.int32(half))

        for j in range(cols_per_tile):
            cl = local * jnp.int32(cols_per_tile) + jnp.int32(j)
            c = jnp.where(is_lo, cl, cl + jnp.int32(32))
            base = c * jnp.int32(b)
            pltpu.sync_copy(idx_hbm.at[pl.ds(base, b)], idx_vs[j])
            g_l = lax.shift_right_logical(cl, jnp.int32(3))
            r = cl & jnp.int32(7)
            lo_base = (g_l * jnp.int32(1 << 23)
                       + r * jnp.int32(1 << _LW_BITS))
            hi_base = g_l * jnp.int32(1 << 23) + r * jnp.int32(1 << _CK_BITS)
            idx_v = idx_vs[j]

            def body_lo(_, o, idx_v=idx_v, lo_base=lo_base):
                v = idx_v[pl.ds(o, lanes)]
                w = lax.shift_right_logical(v, jnp.int32(_LW_BITS))
                idx_v[pl.ds(o, lanes)] = (
                    lax.shift_left(w, jnp.int32(20)) + (v & mask_lw) + lo_base)
                return o + jnp.int32(lanes)

            def body_hi(_, o, idx_v=idx_v, hi_base=hi_base):
                v = idx_v[pl.ds(o, lanes)]
                w = lax.shift_right_logical(v, jnp.int32(_LW_BITS))
                m = lax.shift_right_logical(v, jnp.int32(_CK_BITS)) & jnp.int32(31)
                idx_v[pl.ds(o, lanes)] = (
                    lax.shift_left(w, jnp.int32(20))
                    + lax.shift_left(m, jnp.int32(15))
                    + (v & mask_ck) + hi_base)
                return o + jnp.int32(lanes)

            @pl.when(is_lo)
            def _():
                lax.fori_loop(0, b // lanes, body_lo, jnp.int32(0))

            @pl.when(jnp.logical_not(is_lo))
            def _():
                lax.fori_loop(0, b // lanes, body_hi, jnp.int32(0))

        @pl.when(is_lo)
        def _():
            for j in range(cols_per_tile):
                pltpu.async_copy(lo_hbm.at[idx_vs[j]], val_vs[j], sem)
            for j in range(cols_per_tile):
                pltpu.make_async_copy(lo_hbm.at[idx_vs[j]], val_vs[j],
                                      sem).wait()

        @pl.when(jnp.logical_not(is_lo))
        def _():
            for j in range(cols_per_tile):
                pltpu.async_copy(hi_hbm.at[idx_vs[j]], val_vs[j], sem)
            for j in range(cols_per_tile):
                pltpu.make_async_copy(hi_hbm.at[idx_vs[j]], val_vs[j],
                                      sem).wait()

        for j in range(cols_per_tile):
            cl = local * jnp.int32(cols_per_tile) + jnp.int32(j)
            c = jnp.where(is_lo, cl, cl + jnp.int32(32))
            pltpu.sync_copy(val_vs[j], out_hbm.at[pl.ds(c * jnp.int32(b), b)])

    return gather_kernel(scratch_lo, scratch_hi, idx1d)


def kernel(x, dim, index, sparse_grad):
    del dim, sparse_grad
    n_rows, n_cols = x.shape
    b, c = index.shape
    xt = x.T
    idx1d = index.T.astype(jnp.int32).reshape(-1)
    n_windows = -(-n_rows // _LW)

    scratch_hi = _detile_sc(xt, list(range(4, 8)), n_rows)
    scratch_lo = _detile_tc(xt, list(range(0, 4)), n_windows)
    out1d = _sc_gather(scratch_lo, scratch_hi, idx1d, n_cols, b)
    return out1d.reshape(c, b).T


# speedup vs baseline: 1.0650x; 1.0650x over previous
import functools

import jax
import jax.numpy as jnp
from jax import lax
from jax.experimental import pallas as pl
from jax.experimental.pallas import tpu as pltpu
from jax.experimental.pallas import tpu_sc as plsc

_LW_BITS = 18
_LW = 1 << _LW_BITS
_WIN_BITS = _LW_BITS + 3


def _detile_body(x_ref, o_ref):
    o_ref[...] = x_ref[...].reshape(8 * _LW)


def _detile_tc(xt, n_strips, n_windows):
    return pl.pallas_call(
        _detile_body,
        grid=(n_strips, n_windows),
        in_specs=[pl.BlockSpec((8, _LW), lambda g, w: (g, w))],
        out_specs=pl.BlockSpec((8 * _LW,), lambda g, w: (g * n_windows + w,)),
        out_shape=jax.ShapeDtypeStruct(
            (n_strips * n_windows * 8 * _LW,), jnp.float32),
    )(xt)


def _sc_offsets(idx1d, n_cols, b, n_windows):
    info = plsc.get_sparse_core_info()
    num_workers = info.num_cores * info.num_subcores
    lanes = info.num_lanes
    per_tile = n_cols * b // num_workers

    mesh = plsc.VectorSubcoreMesh(core_axis_name="c", subcore_axis_name="s")

    @functools.partial(
        pl.kernel,
        mesh=mesh,
        out_type=jax.ShapeDtypeStruct((n_cols * b,), jnp.int32),
        scratch_types=[
            pltpu.VMEM((per_tile,), jnp.int32),
        ],
    )
    def offsets_kernel(idx_hbm, out_hbm, idx_v):
        wid = lax.axis_index("s") * info.num_cores + lax.axis_index("c")
        base = wid * jnp.int32(per_tile)
        pltpu.sync_copy(idx_hbm.at[pl.ds(base, per_tile)], idx_v)

        mask = jnp.int32(_LW - 1)
        cb = jnp.int32(b)

        def body(_, o):
            v = idx_v[pl.ds(o, lanes)]
            c = wid * jnp.int32(2) + lax.div(o, cb)
            g = lax.shift_right_logical(c, jnp.int32(3))
            r = c & jnp.int32(7)
            w = lax.shift_right_logical(v, jnp.int32(_LW_BITS))
            off = ((g * jnp.int32(n_windows) + w) * jnp.int32(1 << _WIN_BITS)
                   + r * jnp.int32(_LW) + (v & mask))
            idx_v[pl.ds(o, lanes)] = off
            return o + jnp.int32(lanes)

        lax.fori_loop(0, per_tile // lanes, body, jnp.int32(0))
        pltpu.sync_copy(idx_v, out_hbm.at[pl.ds(base, per_tile)])

    return offsets_kernel(idx1d)


def _sc_gather(scratch, offs1d, n_cols, b):
    info = plsc.get_sparse_core_info()
    num_workers = info.num_cores * info.num_subcores
    cols_per_tile = n_cols // num_workers

    mesh = plsc.VectorSubcoreMesh(core_axis_name="c", subcore_axis_name="s")

    @functools.partial(
        pl.kernel,
        mesh=mesh,
        out_type=jax.ShapeDtypeStruct((n_cols * b,), jnp.float32),
        scratch_types=(
            [pltpu.VMEM((b,), jnp.int32) for _ in range(cols_per_tile)]
            + [pltpu.VMEM((b,), jnp.float32) for _ in range(cols_per_tile)]
            + [pltpu.SemaphoreType.DMA, pltpu.SemaphoreType.DMA]
        ),
    )
    def gather_kernel(scratch_hbm, offs_hbm, out_hbm, *refs):
        idx_vs = refs[:cols_per_tile]
        val_vs = refs[cols_per_tile:2 * cols_per_tile]
        sem_i, sem_g = refs[2 * cols_per_tile:]
        wid = lax.axis_index("s") * info.num_cores + lax.axis_index("c")

        def col_base(j):
            c = wid * jnp.int32(cols_per_tile) + jnp.int32(j)
            return c * jnp.int32(b)

        for j in range(cols_per_tile):
            pltpu.async_copy(offs_hbm.at[pl.ds(col_base(j), b)], idx_vs[j],
                             sem_i)
        for j in range(cols_per_tile):
            pltpu.make_async_copy(offs_hbm.at[pl.ds(col_base(j), b)],
                                  idx_vs[j], sem_i).wait()
            pltpu.async_copy(scratch_hbm.at[idx_vs[j]], val_vs[j], sem_g)
        for j in range(cols_per_tile):
            pltpu.make_async_copy(scratch_hbm.at[idx_vs[j]], val_vs[j],
                                  sem_g).wait()
            pltpu.sync_copy(val_vs[j], out_hbm.at[pl.ds(col_base(j), b)])

    return gather_kernel(scratch, offs1d)


def kernel(x, dim, index, sparse_grad):
    del dim, sparse_grad
    n_rows, n_cols = x.shape
    b, c = index.shape
    xt = x.T
    idx1d = index.T.astype(jnp.int32).reshape(-1)
    n_strips = n_cols // 8
    n_windows = -(-n_rows // _LW)

    offs1d = _sc_offsets(idx1d, n_cols, b, n_windows)
    scratch = _detile_tc(xt, n_strips, n_windows)
    out1d = _sc_gather(scratch, offs1d, n_cols, b)
    return out1d.reshape(c, b).T


# speedup vs baseline: 1.0844x; 1.0183x over previous
import functools

import jax
import jax.numpy as jnp
from jax import lax
from jax.experimental import pallas as pl
from jax.experimental.pallas import tpu as pltpu
from jax.experimental.pallas import tpu_sc as plsc

_LW_BITS = 18
_LW = 1 << _LW_BITS
_WIN_BITS = _LW_BITS + 3


def _detile_body(x_ref, o_ref):
    o_ref[...] = x_ref[...].reshape(8 * _LW)


def _detile_tc(xt, base, n_strips, n_windows):
    return pl.pallas_call(
        _detile_body,
        grid=(n_strips, n_windows),
        in_specs=[pl.BlockSpec((8, _LW),
                               lambda g, w: (g + jnp.int32(base), w))],
        out_specs=pl.BlockSpec((8 * _LW,), lambda g, w: (g * n_windows + w,)),
        out_shape=jax.ShapeDtypeStruct(
            (n_strips * n_windows * 8 * _LW,), jnp.float32),
    )(xt)


def _sc_offsets(idx1d, n_cols, b, n_windows):
    info = plsc.get_sparse_core_info()
    num_workers = info.num_cores * info.num_subcores
    lanes = info.num_lanes
    per_tile = n_cols * b // num_workers

    mesh = plsc.VectorSubcoreMesh(core_axis_name="c", subcore_axis_name="s")

    @functools.partial(
        pl.kernel,
        mesh=mesh,
        out_type=jax.ShapeDtypeStruct((n_cols * b,), jnp.int32),
        scratch_types=[
            pltpu.VMEM((per_tile,), jnp.int32),
        ],
    )
    def offsets_kernel(idx_hbm, out_hbm, idx_v):
        wid = lax.axis_index("s") * info.num_cores + lax.axis_index("c")
        base = wid * jnp.int32(per_tile)
        pltpu.sync_copy(idx_hbm.at[pl.ds(base, per_tile)], idx_v)

        mask = jnp.int32(_LW - 1)
        cb = jnp.int32(b)

        def body(_, o):
            v = idx_v[pl.ds(o, lanes)]
            c = wid * jnp.int32(2) + lax.div(o, cb)
            g = lax.shift_right_logical(c, jnp.int32(3))
            r = c & jnp.int32(7)
            w = lax.shift_right_logical(v, jnp.int32(_LW_BITS))
            off = ((g * jnp.int32(n_windows) + w) * jnp.int32(1 << _WIN_BITS)
                   + r * jnp.int32(_LW) + (v & mask))
            idx_v[pl.ds(o, lanes)] = off
            return o + jnp.int32(lanes)

        lax.fori_loop(0, per_tile // lanes, body, jnp.int32(0))
        pltpu.sync_copy(idx_v, out_hbm.at[pl.ds(base, per_tile)])

    return offsets_kernel(idx1d)


def _sc_gather_half(scratch, offs1d, half, n_cols, b, base_off):
    info = plsc.get_sparse_core_info()
    num_workers = info.num_cores * info.num_subcores

    mesh = plsc.VectorSubcoreMesh(core_axis_name="c", subcore_axis_name="s")

    @functools.partial(
        pl.kernel,
        mesh=mesh,
        out_type=jax.ShapeDtypeStruct((num_workers * b,), jnp.float32),
        scratch_types=[
            pltpu.VMEM((b,), jnp.int32),
            pltpu.VMEM((b,), jnp.float32),
            pltpu.SemaphoreType.DMA,
        ],
    )
    def gather_kernel(scratch_hbm, offs_hbm, out_hbm, idx_v, val_v, sem):
        wid = lax.axis_index("s") * info.num_cores + lax.axis_index("c")
        c = wid + jnp.int32(half * num_workers)
        pltpu.sync_copy(offs_hbm.at[pl.ds(c * jnp.int32(b), b)], idx_v)
        if base_off:
            def body(_, o):
                idx_v[pl.ds(o, 16)] = (idx_v[pl.ds(o, 16)]
                                       - jnp.int32(base_off))
                return o + jnp.int32(16)
            lax.fori_loop(0, b // 16, body, jnp.int32(0))
        pltpu.async_copy(scratch_hbm.at[idx_v], val_v, sem).wait()
        pltpu.sync_copy(val_v, out_hbm.at[pl.ds(wid * jnp.int32(b), b)])

    return gather_kernel(scratch, offs1d)


def kernel(x, dim, index, sparse_grad):
    del dim, sparse_grad
    n_rows, n_cols = x.shape
    b, c = index.shape
    xt = x.T
    idx1d = index.T.astype(jnp.int32).reshape(-1)
    n_strips = n_cols // 8
    n_windows = -(-n_rows // _LW)

    offs1d = _sc_offsets(idx1d, n_cols, b, n_windows)
    half_words = (n_strips // 2) * n_windows * 8 * _LW
    s0 = _detile_tc(xt, 0, n_strips // 2, n_windows)
    o0 = _sc_gather_half(s0, offs1d, 0, n_cols, b, 0)
    s1 = _detile_tc(xt, n_strips // 2, n_strips // 2, n_windows)
    o1 = _sc_gather_half(s1, offs1d, 1, n_cols, b, half_words)
    out1d = jnp.concatenate([o0, o1])
    return out1d.reshape(c, b).T


# speedup vs baseline: 1.0873x; 1.0026x over previous
import functools

import jax
import jax.numpy as jnp
from jax import lax
from jax.experimental import pallas as pl
from jax.experimental.pallas import tpu as pltpu
from jax.experimental.pallas import tpu_sc as plsc

_LW_BITS = 18
_LW = 1 << _LW_BITS
_WIN_BITS = _LW_BITS + 3


def _detile_body(x_ref, o_ref):
    o_ref[...] = x_ref[...].reshape(8 * _LW)


def _detile_tc(xt, base, n_strips, n_windows):
    return pl.pallas_call(
        _detile_body,
        grid=(n_strips, n_windows),
        in_specs=[pl.BlockSpec((8, _LW),
                               lambda g, w: (g + jnp.int32(base), w))],
        out_specs=pl.BlockSpec((8 * _LW,), lambda g, w: (g * n_windows + w,)),
        out_shape=jax.ShapeDtypeStruct(
            (n_strips * n_windows * 8 * _LW,), jnp.float32),
    )(xt)


def _sc_offsets(idx1d, n_cols, b, n_windows):
    info = plsc.get_sparse_core_info()
    num_workers = info.num_cores * info.num_subcores
    lanes = info.num_lanes
    per_tile = n_cols * b // num_workers

    mesh = plsc.VectorSubcoreMesh(core_axis_name="c", subcore_axis_name="s")

    @functools.partial(
        pl.kernel,
        mesh=mesh,
        out_type=jax.ShapeDtypeStruct((n_cols * b,), jnp.int32),
        scratch_types=[
            pltpu.VMEM((per_tile,), jnp.int32),
        ],
    )
    def offsets_kernel(idx_hbm, out_hbm, idx_v):
        wid = lax.axis_index("s") * info.num_cores + lax.axis_index("c")
        base = wid * jnp.int32(per_tile)
        pltpu.sync_copy(idx_hbm.at[pl.ds(base, per_tile)], idx_v)

        mask = jnp.int32(_LW - 1)
        cb = jnp.int32(b)

        def body(_, o):
            v = idx_v[pl.ds(o, lanes)]
            c = wid * jnp.int32(2) + lax.div(o, cb)
            g = lax.shift_right_logical(c, jnp.int32(3))
            r = c & jnp.int32(7)
            w = lax.shift_right_logical(v, jnp.int32(_LW_BITS))
            off = ((g * jnp.int32(n_windows) + w) * jnp.int32(1 << _WIN_BITS)
                   + r * jnp.int32(_LW) + (v & mask))
            idx_v[pl.ds(o, lanes)] = off
            return o + jnp.int32(lanes)

        lax.fori_loop(0, per_tile // lanes, body, jnp.int32(0))
        pltpu.sync_copy(idx_v, out_hbm.at[pl.ds(base, per_tile)])

    return offsets_kernel(idx1d)


def _sc_gather_part(scratch, offs1d, part, n_part_cols, b, base_off):
    info = plsc.get_sparse_core_info()
    num_workers = info.num_cores * info.num_subcores
    per_tile = n_part_cols * b // num_workers

    mesh = plsc.VectorSubcoreMesh(core_axis_name="c", subcore_axis_name="s")

    @functools.partial(
        pl.kernel,
        mesh=mesh,
        out_type=jax.ShapeDtypeStruct((n_part_cols * b,), jnp.float32),
        scratch_types=[
            pltpu.VMEM((per_tile,), jnp.int32),
            pltpu.VMEM((per_tile,), jnp.float32),
            pltpu.SemaphoreType.DMA,
        ],
    )
    def gather_kernel(scratch_hbm, offs_hbm, out_hbm, idx_v, val_v, sem):
        wid = lax.axis_index("s") * info.num_cores + lax.axis_index("c")
        gbase = (jnp.int32(part * n_part_cols * b)
                 + wid * jnp.int32(per_tile))
        pltpu.sync_copy(offs_hbm.at[pl.ds(gbase, per_tile)], idx_v)
        if base_off:
            def body(_, o):
                idx_v[pl.ds(o, 16)] = (idx_v[pl.ds(o, 16)]
                                       - jnp.int32(base_off))
                return o + jnp.int32(16)
            lax.fori_loop(0, per_tile // 16, body, jnp.int32(0))
        pltpu.async_copy(scratch_hbm.at[idx_v], val_v, sem).wait()
        pltpu.sync_copy(val_v,
                        out_hbm.at[pl.ds(wid * jnp.int32(per_tile),
                                         per_tile)])

    return gather_kernel(scratch, offs1d)


def kernel(x, dim, index, sparse_grad):
    del dim, sparse_grad
    n_rows, n_cols = x.shape
    b, c = index.shape
    xt = x.T
    idx1d = index.T.astype(jnp.int32).reshape(-1)
    n_strips = n_cols // 8
    n_windows = -(-n_rows // _LW)

    offs1d = _sc_offsets(idx1d, n_cols, b, n_windows)
    n_parts = 4
    strips_per_part = n_strips // n_parts
    part_words = strips_per_part * n_windows * 8 * _LW
    outs = []
    for p in range(n_parts):
        sp = _detile_tc(xt, p * strips_per_part, strips_per_part,
                        n_windows)
        outs.append(_sc_gather_part(sp, offs1d, p, strips_per_part * 8, b,
                                    p * part_words))
    out1d = jnp.concatenate(outs)
    return out1d.reshape(c, b).T
